# baseline (speedup 1.0000x reference)
import functools

import jax
import jax.numpy as jnp
from jax import lax
from jax.experimental import pallas as pl
from jax.experimental.pallas import tpu as pltpu
from jax.experimental.pallas import tpu_sc as plsc

_N = 10000
_E = 320000
_D = 128
_NORM = 100.0

_TE = 2560
_TN = 2000
_NW = 32
_K = 80

_CH0 = 63
_CH1 = 62
_EPW0 = _CH0 * _K
_EPW1 = _CH1 * _K
_EH0 = _NW * _EPW0
_EH1 = _NW * _EPW1
_T0 = _EH0 // _TE
_T1 = _EH1 // _TE

_F32 = jnp.float32


def _silu(v):
    return v / (1.0 + jnp.exp(-v))


def _abp_body(h, wa, wb, pp, a, b):
    hv = h[...]
    ppv = pp[...]
    a[...] = jnp.concatenate(
        [jnp.dot(hv, wa[...], preferred_element_type=_F32), ppv], axis=1)
    b[...] = jnp.concatenate(
        [jnp.dot(hv, wb[...], preferred_element_type=_F32), ppv], axis=1)


def _tc_abp(h, wa, wb, posp):
    bs_n = pl.BlockSpec((_TN, _D), lambda i: (i, 0))
    bs_o = pl.BlockSpec((_TN, 2 * _D), lambda i: (i, 0))
    bs_w = pl.BlockSpec((_D, _D), lambda i: (0, 0))
    return pl.pallas_call(
        _abp_body,
        grid=(_N // _TN,),
        in_specs=[bs_n, bs_w, bs_w, bs_n],
        out_specs=[bs_o, bs_o],
        out_shape=[jax.ShapeDtypeStruct((_N, 2 * _D), _F32)] * 2,
    )(h, wa, wb, posp)


def _edge_body(g, ea, df, wce, wcr, b1, w2, b2, out):
    d = df[...]
    radial = jnp.sum(d * d, axis=1, keepdims=True)
    c = jnp.dot(ea[...], wce[...], preferred_element_type=_F32)
    m = _silu(g[...] + c + radial * wcr[...] + b1[...])
    out[...] = _silu(jnp.dot(m, w2[...], preferred_element_type=_F32) + b2[...])


def _tc_edge(g, ea, df, wce, wcr, b1, w2, b2, toff, ntile):
    bs_g = pl.BlockSpec((_TE, _D), lambda i: (i, 0))
    bs_ea = pl.BlockSpec((_TE, 16), lambda i: (i + toff, 0))
    bs_16 = pl.BlockSpec((_TE, 16), lambda i: (i, 0))
    bw = lambda s: pl.BlockSpec(s, lambda i: (0, 0))
    return pl.pallas_call(
        _edge_body,
        grid=(ntile,),
        in_specs=[bs_g, bs_ea, bs_16, bw((16, _D)), bw((1, _D)), bw((1, _D)),
                  bw((_D, _D)), bw((1, _D))],
        out_specs=bs_g,
        out_shape=jax.ShapeDtypeStruct((ntile * _TE, _D), _F32),
    )(g, ea, df, wce, wcr, b1, w2, b2)


def _coord_body(g, ea, df, wce, wcr, b1, w2, b2, w3, out):
    d = df[...]
    radial = jnp.sum(d * d, axis=1, keepdims=True)
    c = jnp.dot(ea[...], wce[...], preferred_element_type=_F32)
    t = _silu(g[...] + c + radial * wcr[...] + b1[...])
    t = _silu(jnp.dot(t, w2[...], preferred_element_type=_F32) + b2[...])
    phi = jnp.dot(t, w3[...], preferred_element_type=_F32)[:, 0:1]
    cd = d / (jnp.sqrt(radial + 1e-8) + 1.0)
    out[...] = jnp.concatenate([cd * phi, jnp.zeros((_TE, _D - 16), _F32)], axis=1)


def _tc_coord(g, ea, df, wce, wcr, b1, w2, b2, w3, toff, ntile):
    bs_g = pl.BlockSpec((_TE, _D), lambda i: (i, 0))
    bs_ea = pl.BlockSpec((_TE, 16), lambda i: (i + toff, 0))
    bs_16 = pl.BlockSpec((_TE, 16), lambda i: (i, 0))
    bw = lambda s: pl.BlockSpec(s, lambda i: (0, 0))
    return pl.pallas_call(
        _coord_body,
        grid=(ntile,),
        in_specs=[bs_g, bs_ea, bs_16, bw((16, _D)), bw((1, _D)), bw((1, _D)),
                  bw((_D, _D)), bw((1, _D)), bw((_D, 8))],
        out_specs=bs_g,
        out_shape=jax.ShapeDtypeStruct((ntile * _TE, _D), _F32),
    )(g, ea, df, wce, wcr, b1, w2, b2, w3)


def _node_body(h, p0, p1, p2, p3, w1h, w1a, b1, w2, b2, wa, wb, hn, a, b):
    hv = h[...]
    agg = (p0[...] + p1[...] + p2[...] + p3[...]) * (1.0 / _NORM)
    u = _silu(jnp.dot(hv, w1h[...], preferred_element_type=_F32)
              + jnp.dot(agg, w1a[...], preferred_element_type=_F32) + b1[...])
    hnv = hv + jnp.dot(u, w2[...], preferred_element_type=_F32) + b2[...]
    hn[...] = hnv
    a[...] = jnp.dot(hnv, wa[...], preferred_element_type=_F32)
    b[...] = jnp.dot(hnv, wb[...], preferred_element_type=_F32)


def _tc_node(h, p0, p1, p2, p3, w1h, w1a, b1, w2, b2, wa, wb):
    bs_n = pl.BlockSpec((_TN, _D), lambda i: (i, 0))
    bw = lambda s: pl.BlockSpec(s, lambda i: (0, 0))
    return pl.pallas_call(
        _node_body,
        grid=(_N // _TN,),
        in_specs=[bs_n, bs_n, bs_n, bs_n, bs_n, bw((_D, _D)), bw((_D, _D)),
                  bw((1, _D)), bw((_D, _D)), bw((1, _D)), bw((_D, _D)),
                  bw((_D, _D))],
        out_specs=[bs_n, bs_n, bs_n],
        out_shape=[jax.ShapeDtypeStruct((_N, _D), _F32)] * 3,
    )(h, p0, p1, p2, p3, w1h, w1a, b1, w2, b2, wa, wb)


def _pos_body(posp, q0, q1, q2, q3, mask, out):
    q = (q0[...] + q1[...] + q2[...] + q3[...]) * (1.0 / _NORM)
    out[...] = (posp[...] + q) * mask[...]


def _tc_pos(posp, q0, q1, q2, q3, mask):
    bs16 = pl.BlockSpec((_TN, 16), lambda i: (i, 0))
    bs1 = pl.BlockSpec((_TN, 1), lambda i: (i, 0))
    return pl.pallas_call(
        _pos_body,
        grid=(_N // _TN,),
        in_specs=[bs16, bs16, bs16, bs16, bs16, bs1],
        out_specs=bs16,
        out_shape=jax.ShapeDtypeStruct((_N, 16), _F32),
    )(posp, q0, q1, q2, q3, mask)


def _sc_mesh():
    return plsc.VectorSubcoreMesh(core_axis_name="c", subcore_axis_name="s")


def _pipeline(nchunk, load, drain, process):
    load(0, 0)
    if nchunk % 2 == 1:
        def pair(j, carry):
            c0 = j * 2
            load(c0 + 1, 1)
            drain(0)
            process(c0, 0)
            load(c0 + 2, 0)
            drain(1)
            process(c0 + 1, 1)
            return carry

        lax.fori_loop(0, (nchunk - 1) // 2, pair, 0)
        drain(0)
        process(nchunk - 1, 0)
    else:
        def pair(j, carry):
            c0 = j * 2
            load(c0 + 1, 1)
            drain(0)
            process(c0, 0)
            load(c0 + 2, 0)
            drain(1)
            process(c0 + 1, 1)
            return carry

        lax.fori_loop(0, nchunk // 2 - 1, pair, 0)
        load(nchunk - 1, 1)
        drain(0)
        process(nchunk - 2, 0)
        drain(1)
        process(nchunk - 1, 1)


def _pipeline3(nchunk, load_idx, wait_idx, issue, drain, process):
    load_idx(0, 0)
    wait_idx(0)
    issue(0)
    load_idx(1, 1)

    def pair(j, carry):
        c0 = j * 2
        wait_idx(1)
        issue(1)
        drain(0)
        load_idx(c0 + 2, 0)
        process(c0, 0)
        wait_idx(0)
        issue(0)
        drain(1)
        load_idx(c0 + 3, 1)
        process(c0 + 1, 1)
        return carry

    if nchunk % 2 == 1:
        lax.fori_loop(0, (nchunk - 1) // 2, pair, 0)
        wait_idx(1)
        drain(0)
        process(nchunk - 1, 0)
    else:
        lax.fori_loop(0, nchunk // 2 - 1, pair, 0)
        wait_idx(1)
        issue(1)
        drain(0)
        process(nchunk - 2, 0)
        drain(1)
        process(nchunk - 1, 1)


def _sc_gather_pos(ap, bp, row, col, ebase, epw, nchunk):
    ne = _NW * epw

    @functools.partial(
        pl.kernel, mesh=_sc_mesh(),
        out_type=[jax.ShapeDtypeStruct((ne, _D), _F32),
                  jax.ShapeDtypeStruct((ne, 16), _F32)],
        scratch_types=[pltpu.VMEM((_K,), jnp.int32), pltpu.VMEM((_K,), jnp.int32),
                       pltpu.VMEM((_K,), jnp.int32), pltpu.VMEM((_K,), jnp.int32),
                       pltpu.VMEM((_K, 2 * _D), _F32), pltpu.VMEM((_K, 2 * _D), _F32),
                       pltpu.VMEM((_K, 2 * _D), _F32), pltpu.VMEM((_K, 2 * _D), _F32),
                       pltpu.VMEM((_K, _D), _F32), pltpu.VMEM((_K, 16), _F32),
                       pltpu.SemaphoreType.DMA, pltpu.SemaphoreType.DMA,
                       pltpu.SemaphoreType.DMA, pltpu.SemaphoreType.DMA],
    )
    def k(a_hbm, b_hbm, row_hbm, col_hbm, g_out, d_out,
          ir0, ic0, ir1, ic1, ba0, bb0, ba1, bb1, og, pd,
          sem0, sem1, semi0, semi1):
        wid = lax.axis_index("s") * 2 + lax.axis_index("c")
        lbase0 = wid * epw
        slots = ((ir0, ic0, ba0, bb0, sem0, semi0),
                 (ir1, ic1, ba1, bb1, sem1, semi1))

        def load_idx(chunk, slot):
            ir, ic, ba, bb, sem, semi = slots[slot]
            cc = jnp.minimum(chunk, nchunk - 1)
            base = ebase + lbase0 + cc * _K
            pltpu.async_copy(row_hbm.at[pl.ds(base, _K)], ir, semi)
            pltpu.async_copy(col_hbm.at[pl.ds(base, _K)], ic, semi)

        def wait_idx(slot):
            ir, ic, ba, bb, sem, semi = slots[slot]
            pltpu.make_async_copy(row_hbm.at[pl.ds(0, _K)], ir, semi).wait()
            pltpu.make_async_copy(col_hbm.at[pl.ds(0, _K)], ic, semi).wait()

        def issue(slot):
            ir, ic, ba, bb, sem, semi = slots[slot]
            pltpu.async_copy(a_hbm.at[ir], ba, sem)
            pltpu.async_copy(b_hbm.at[ic], bb, sem)

        def drain(slot):
            ir, ic, ba, bb, sem, semi = slots[slot]
            pltpu.make_async_copy(a_hbm.at[ir], ba, sem).wait()
            pltpu.make_async_copy(b_hbm.at[ic], bb, sem).wait()

        def process(chunk, slot):
            ir, ic, ba, bb, sem, semi = slots[slot]
            lbase = lbase0 + chunk * _K

            def addrow(r, cc):
                for l in range(_D // 16):
                    sl = pl.ds(l * 16, 16)
                    og[r, sl] = ba[r, sl] + bb[r, sl]
                sp = pl.ds(_D, 16)
                s16 = pl.ds(0, 16)
                pd[r, s16] = ba[r, sp] - bb[r, sp]
                return cc

            lax.fori_loop(0, _K, addrow, 0)
            pltpu.sync_copy(og, g_out.at[pl.ds(lbase, _K)])
            pltpu.sync_copy(pd, d_out.at[pl.ds(lbase, _K)])

        _pipeline3(nchunk, load_idx, wait_idx, issue, drain, process)

    return k(ap, bp, row, col)


def _sc_gather(a, b, row, col, ebase, epw, nchunk):
    ne = _NW * epw

    @functools.partial(
        pl.kernel, mesh=_sc_mesh(),
        out_type=jax.ShapeDtypeStruct((ne, _D), _F32),
        scratch_types=[pltpu.VMEM((_K,), jnp.int32), pltpu.VMEM((_K,), jnp.int32),
                       pltpu.VMEM((_K,), jnp.int32), pltpu.VMEM((_K,), jnp.int32),
                       pltpu.VMEM((_K, _D), _F32), pltpu.VMEM((_K, _D), _F32),
                       pltpu.VMEM((_K, _D), _F32), pltpu.VMEM((_K, _D), _F32),
                       pltpu.SemaphoreType.DMA, pltpu.SemaphoreType.DMA,
                       pltpu.SemaphoreType.DMA, pltpu.SemaphoreType.DMA],
    )
    def k(a_hbm, b_hbm, row_hbm, col_hbm, g_out,
          ir0, ic0, ir1, ic1, ba0, bb0, ba1, bb1, sem0, sem1, semi0, semi1):
        wid = lax.axis_index("s") * 2 + lax.axis_index("c")
        lbase0 = wid * epw
        slots = ((ir0, ic0, ba0, bb0, sem0, semi0),
                 (ir1, ic1, ba1, bb1, sem1, semi1))

        def load_idx(chunk, slot):
            ir, ic, ba, bb, sem, semi = slots[slot]
            cc = jnp.minimum(chunk, nchunk - 1)
            base = ebase + lbase0 + cc * _K
            pltpu.async_copy(row_hbm.at[pl.ds(base, _K)], ir, semi)
            pltpu.async_copy(col_hbm.at[pl.ds(base, _K)], ic, semi)

        def wait_idx(slot):
            ir, ic, ba, bb, sem, semi = slots[slot]
            pltpu.make_async_copy(row_hbm.at[pl.ds(0, _K)], ir, semi).wait()
            pltpu.make_async_copy(col_hbm.at[pl.ds(0, _K)], ic, semi).wait()

        def issue(slot):
            ir, ic, ba, bb, sem, semi = slots[slot]
            pltpu.async_copy(a_hbm.at[ir], ba, sem)
            pltpu.async_copy(b_hbm.at[ic], bb, sem)

        def drain(slot):
            ir, ic, ba, bb, sem, semi = slots[slot]
            pltpu.make_async_copy(a_hbm.at[ir], ba, sem).wait()
            pltpu.make_async_copy(b_hbm.at[ic], bb, sem).wait()

        def process(chunk, slot):
            ir, ic, ba, bb, sem, semi = slots[slot]
            lbase = lbase0 + chunk * _K

            def addrow(r, cc):
                for l in range(_D // 16):
                    sl = pl.ds(l * 16, 16)
                    ba[r, sl] = ba[r, sl] + bb[r, sl]
                return cc

            lax.fori_loop(0, _K, addrow, 0)
            pltpu.sync_copy(ba, g_out.at[pl.ds(lbase, _K)])

        _pipeline3(nchunk, load_idx, wait_idx, issue, drain, process)

    return k(a, b, row, col)


def _sc_scatter(m, row, zeros, ebase, epw, nchunk):
    @functools.partial(
        pl.kernel, mesh=_sc_mesh(),
        out_type=jax.ShapeDtypeStruct((2, _N, _D), _F32),
        scratch_types=[pltpu.VMEM((_K,), jnp.int32), pltpu.VMEM((_K,), jnp.int32),
                       pltpu.VMEM((_K, _D), _F32), pltpu.VMEM((_K, _D), _F32),
                       pltpu.VMEM_SHARED((_N, _D), _F32),
                       pltpu.SemaphoreType.DMA, pltpu.SemaphoreType.DMA],
    )
    def k(m_hbm, row_hbm, z_hbm, out_hbm, idx0, idx1, buf0, buf1, acc,
          sem0, sem1):
        c = lax.axis_index("c")
        s = lax.axis_index("s")
        wid = s * 2 + c
        a0 = s * 624
        pltpu.sync_copy(z_hbm.at[pl.ds(a0, 640)], acc.at[pl.ds(a0, 640)])
        plsc.subcore_barrier()
        lbase0 = wid * epw
        slots = ((idx0, buf0, sem0), (idx1, buf1, sem1))

        def load(chunk, slot):
            idx, buf, sem = slots[slot]
            lbase = lbase0 + chunk * _K
            pltpu.async_copy(row_hbm.at[pl.ds(ebase + lbase, _K)], idx, sem)
            pltpu.async_copy(m_hbm.at[pl.ds(lbase, _K)], buf, sem)

        def drain(slot):
            idx, buf, sem = slots[slot]
            pltpu.make_async_copy(row_hbm.at[pl.ds(0, _K)], idx, sem).wait()
            pltpu.make_async_copy(m_hbm.at[pl.ds(0, _K)], buf, sem).wait()

        def process(chunk, slot):
            idx, buf, sem = slots[slot]
            pltpu.sync_copy(buf, acc.at[idx], add=True)

        _pipeline(nchunk, load, drain, process)
        plsc.subcore_barrier()
        pltpu.sync_copy(acc.at[pl.ds(a0, 640)],
                        out_hbm.at[c, pl.ds(a0, 640)])

    return k(m, row, zeros)


def kernel(x, pos, edge_index, mask, edge_attr, params):
    row = edge_index[0]
    col = edge_index[1]
    posp = jnp.pad(pos, ((0, 0), (0, _D - 3)))
    posp16 = posp[:, :16]
    zeros_d = jnp.zeros((_N, _D), _F32)

    def split(w):
        return w[:_D], w[_D:2 * _D], w[2 * _D:2 * _D + 1], w[2 * _D + 1:]

    def r1(v):
        return v[None, :]

    g0p = params['gcl0']
    g1p = params['gcl1']
    egp = params['egcl']
    wa0, wb0, wcr0, wce0 = split(g0p['ew1'])
    wa1, wb1, wcr1, wce1 = split(g1p['ew1'])
    wa2, wb2, wcr2, wce2 = split(egp['cw1'])
    w3p = jnp.pad(egp['cw3'], ((0, 0), (0, 7)))

    halves = ((0, _EPW0, _CH0, 0, _T0), (_EH0, _EPW1, _CH1, _T0, _T1))

    def edge_layer(a, b, wce, wcr, b1, w2, b2, diffs):
        qs = []
        for (ebase, epw, nchunk, toff, ntile), df in zip(halves, diffs):
            g = _sc_gather(a, b, row, col, ebase, epw, nchunk)
            m = _tc_edge(g, edge_attr, df, wce, wcr, b1, w2, b2, toff, ntile)
            q = _sc_scatter(m, row, zeros_d, ebase, epw, nchunk)
            qs += [q[0], q[1]]
        return qs

    ap0, bp0 = _tc_abp(x, wa0, wb0, posp)
    qs0 = []
    diffs = []
    for ebase, epw, nchunk, toff, ntile in halves:
        g, df = _sc_gather_pos(ap0, bp0, row, col, ebase, epw, nchunk)
        diffs.append(df)
        m = _tc_edge(g, edge_attr, df, wce0, wcr0, r1(g0p['eb1']),
                     g0p['ew2'], r1(g0p['eb2']), toff, ntile)
        q = _sc_scatter(m, row, zeros_d, ebase, epw, nchunk)
        qs0 += [q[0], q[1]]
    h1, a1, b1 = _tc_node(x, *qs0,
                          g0p['nw1'][:_D], g0p['nw1'][_D:], r1(g0p['nb1']),
                          g0p['nw2'], r1(g0p['nb2']), wa1, wb1)

    qs1 = edge_layer(a1, b1, wce1, wcr1, r1(g1p['eb1']),
                     g1p['ew2'], r1(g1p['eb2']), diffs)
    h2, a2, b2 = _tc_node(h1, *qs1,
                          g1p['nw1'][:_D], g1p['nw1'][_D:], r1(g1p['nb1']),
                          g1p['nw2'], r1(g1p['nb2']), wa2, wb2)

    qsp = []
    for (ebase, epw, nchunk, toff, ntile), df in zip(halves, diffs):
        g = _sc_gather(a2, b2, row, col, ebase, epw, nchunk)
        t = _tc_coord(g, edge_attr, df, wce2, wcr2, r1(egp['cb1']),
                      egp['cw2'], r1(egp['cb2']), w3p, toff, ntile)
        q = _sc_scatter(t, row, zeros_d, ebase, epw, nchunk)
        qsp += [q[0, :, :16], q[1, :, :16]]
    pos16 = _tc_pos(posp16, *qsp, mask)
    return h2, pos16[:, :3]


# speedup vs baseline: 1.0599x; 1.0599x over previous
import functools

import jax
import jax.numpy as jnp
from jax import lax
from jax.experimental import pallas as pl
from jax.experimental.pallas import tpu as pltpu
from jax.experimental.pallas import tpu_sc as plsc

_N = 10000
_E = 320000
_D = 128
_NORM = 100.0

_TE = 2560
_TN = 2000
_NW = 32
_K = 80

_CH0 = 63
_CH1 = 62
_EPW0 = _CH0 * _K
_EPW1 = _CH1 * _K
_EH0 = _NW * _EPW0
_EH1 = _NW * _EPW1
_T0 = _EH0 // _TE
_T1 = _EH1 // _TE

_F32 = jnp.float32


def _silu(v):
    return v / (1.0 + jnp.exp(-v))


def _ab_body(h, wa, wb, a, b):
    hv = h[...]
    a[...] = jnp.dot(hv, wa[...], preferred_element_type=_F32)
    b[...] = jnp.dot(hv, wb[...], preferred_element_type=_F32)


def _tc_ab(h, wa, wb):
    bs_n = pl.BlockSpec((_TN, _D), lambda i: (i, 0))
    bs_w = pl.BlockSpec((_D, _D), lambda i: (0, 0))
    return pl.pallas_call(
        _ab_body,
        grid=(_N // _TN,),
        in_specs=[bs_n, bs_w, bs_w],
        out_specs=[bs_n, bs_n],
        out_shape=[jax.ShapeDtypeStruct((_N, _D), _F32)] * 2,
    )(h, wa, wb)


def _edge_body(g, ea, df, wce, wcr, b1, w2, b2, out):
    d = df[...]
    radial = jnp.sum(d * d, axis=1, keepdims=True)
    c = jnp.dot(ea[...], wce[...], preferred_element_type=_F32)
    m = _silu(g[...] + c + radial * wcr[...] + b1[...])
    out[...] = _silu(jnp.dot(m, w2[...], preferred_element_type=_F32) + b2[...])


def _tc_edge(g, ea, df, wce, wcr, b1, w2, b2, toff, ntile):
    bs_g = pl.BlockSpec((_TE, _D), lambda i: (i, 0))
    bs_ea = pl.BlockSpec((_TE, 16), lambda i: (i + toff, 0))
    bs_16 = pl.BlockSpec((_TE, 16), lambda i: (i, 0))
    bw = lambda s: pl.BlockSpec(s, lambda i: (0, 0))
    return pl.pallas_call(
        _edge_body,
        grid=(ntile,),
        in_specs=[bs_g, bs_ea, bs_16, bw((16, _D)), bw((1, _D)), bw((1, _D)),
                  bw((_D, _D)), bw((1, _D))],
        out_specs=bs_g,
        out_shape=jax.ShapeDtypeStruct((ntile * _TE, _D), _F32),
    )(g, ea, df, wce, wcr, b1, w2, b2)


def _coord_body(g, ea, df, wce, wcr, b1, w2, b2, w3, out):
    d = df[...]
    radial = jnp.sum(d * d, axis=1, keepdims=True)
    c = jnp.dot(ea[...], wce[...], preferred_element_type=_F32)
    t = _silu(g[...] + c + radial * wcr[...] + b1[...])
    t = _silu(jnp.dot(t, w2[...], preferred_element_type=_F32) + b2[...])
    phi = jnp.dot(t, w3[...], preferred_element_type=_F32)[:, 0:1]
    cd = d / (jnp.sqrt(radial + 1e-8) + 1.0)
    out[...] = jnp.concatenate([cd * phi, jnp.zeros((_TE, _D - 16), _F32)], axis=1)


def _tc_coord(g, ea, df, wce, wcr, b1, w2, b2, w3, toff, ntile):
    bs_g = pl.BlockSpec((_TE, _D), lambda i: (i, 0))
    bs_ea = pl.BlockSpec((_TE, 16), lambda i: (i + toff, 0))
    bs_16 = pl.BlockSpec((_TE, 16), lambda i: (i, 0))
    bw = lambda s: pl.BlockSpec(s, lambda i: (0, 0))
    return pl.pallas_call(
        _coord_body,
        grid=(ntile,),
        in_specs=[bs_g, bs_ea, bs_16, bw((16, _D)), bw((1, _D)), bw((1, _D)),
                  bw((_D, _D)), bw((1, _D)), bw((_D, 8))],
        out_specs=bs_g,
        out_shape=jax.ShapeDtypeStruct((ntile * _TE, _D), _F32),
    )(g, ea, df, wce, wcr, b1, w2, b2, w3)


def _node_body(h, p0, p1, p2, p3, w1h, w1a, b1, w2, b2, wa, wb, hn, a, b):
    hv = h[...]
    agg = (p0[...] + p1[...] + p2[...] + p3[...]) * (1.0 / _NORM)
    u = _silu(jnp.dot(hv, w1h[...], preferred_element_type=_F32)
              + jnp.dot(agg, w1a[...], preferred_element_type=_F32) + b1[...])
    hnv = hv + jnp.dot(u, w2[...], preferred_element_type=_F32) + b2[...]
    hn[...] = hnv
    a[...] = jnp.dot(hnv, wa[...], preferred_element_type=_F32)
    b[...] = jnp.dot(hnv, wb[...], preferred_element_type=_F32)


def _tc_node(h, p0, p1, p2, p3, w1h, w1a, b1, w2, b2, wa, wb):
    bs_n = pl.BlockSpec((_TN, _D), lambda i: (i, 0))
    bw = lambda s: pl.BlockSpec(s, lambda i: (0, 0))
    return pl.pallas_call(
        _node_body,
        grid=(_N // _TN,),
        in_specs=[bs_n, bs_n, bs_n, bs_n, bs_n, bw((_D, _D)), bw((_D, _D)),
                  bw((1, _D)), bw((_D, _D)), bw((1, _D)), bw((_D, _D)),
                  bw((_D, _D))],
        out_specs=[bs_n, bs_n, bs_n],
        out_shape=[jax.ShapeDtypeStruct((_N, _D), _F32)] * 3,
    )(h, p0, p1, p2, p3, w1h, w1a, b1, w2, b2, wa, wb)


def _pos_body(posp, q0, q1, q2, q3, mask, out):
    q = (q0[...] + q1[...] + q2[...] + q3[...]) * (1.0 / _NORM)
    out[...] = (posp[...] + q) * mask[...]


def _tc_pos(posp, q0, q1, q2, q3, mask):
    bs16 = pl.BlockSpec((_TN, 16), lambda i: (i, 0))
    bs1 = pl.BlockSpec((_TN, 1), lambda i: (i, 0))
    return pl.pallas_call(
        _pos_body,
        grid=(_N // _TN,),
        in_specs=[bs16, bs16, bs16, bs16, bs16, bs1],
        out_specs=bs16,
        out_shape=jax.ShapeDtypeStruct((_N, 16), _F32),
    )(posp, q0, q1, q2, q3, mask)


def _sc_mesh():
    return plsc.VectorSubcoreMesh(core_axis_name="c", subcore_axis_name="s")


def _pipeline(nchunk, load, drain, process):
    load(0, 0)
    if nchunk % 2 == 1:
        def pair(j, carry):
            c0 = j * 2
            load(c0 + 1, 1)
            drain(0)
            process(c0, 0)
            load(c0 + 2, 0)
            drain(1)
            process(c0 + 1, 1)
            return carry

        lax.fori_loop(0, (nchunk - 1) // 2, pair, 0)
        drain(0)
        process(nchunk - 1, 0)
    else:
        def pair(j, carry):
            c0 = j * 2
            load(c0 + 1, 1)
            drain(0)
            process(c0, 0)
            load(c0 + 2, 0)
            drain(1)
            process(c0 + 1, 1)
            return carry

        lax.fori_loop(0, nchunk // 2 - 1, pair, 0)
        load(nchunk - 1, 1)
        drain(0)
        process(nchunk - 2, 0)
        drain(1)
        process(nchunk - 1, 1)


def _pipeline3(nchunk, load_idx, wait_idx, issue, drain, process):
    load_idx(0, 0)
    wait_idx(0)
    issue(0)
    load_idx(1, 1)

    def pair(j, carry):
        c0 = j * 2
        wait_idx(1)
        issue(1)
        drain(0)
        load_idx(c0 + 2, 0)
        process(c0, 0)
        wait_idx(0)
        issue(0)
        drain(1)
        load_idx(c0 + 3, 1)
        process(c0 + 1, 1)
        return carry

    if nchunk % 2 == 1:
        lax.fori_loop(0, (nchunk - 1) // 2, pair, 0)
        wait_idx(1)
        drain(0)
        process(nchunk - 1, 0)
    else:
        lax.fori_loop(0, nchunk // 2 - 1, pair, 0)
        wait_idx(1)
        issue(1)
        drain(0)
        process(nchunk - 2, 0)
        drain(1)
        process(nchunk - 1, 1)


def _sc_gather_pos(a, b, row, col, posp, ebase, epw, nchunk):
    ne = _NW * epw

    @functools.partial(
        pl.kernel, mesh=_sc_mesh(),
        out_type=[jax.ShapeDtypeStruct((ne, _D), _F32),
                  jax.ShapeDtypeStruct((ne, 16), _F32)],
        scratch_types=[pltpu.VMEM((_K,), jnp.int32), pltpu.VMEM((_K,), jnp.int32),
                       pltpu.VMEM((_K,), jnp.int32), pltpu.VMEM((_K,), jnp.int32),
                       pltpu.VMEM((_K, _D), _F32), pltpu.VMEM((_K, _D), _F32),
                       pltpu.VMEM((_K, _D), _F32), pltpu.VMEM((_K, _D), _F32),
                       pltpu.VMEM((_K, _D), _F32), pltpu.VMEM((_K, _D), _F32),
                       pltpu.VMEM((_K, _D), _F32), pltpu.VMEM((_K, _D), _F32),
                       pltpu.VMEM((_K, 16), _F32),
                       pltpu.SemaphoreType.DMA, pltpu.SemaphoreType.DMA,
                       pltpu.SemaphoreType.DMA, pltpu.SemaphoreType.DMA],
    )
    def k(a_hbm, b_hbm, row_hbm, col_hbm, p_hbm, g_out, d_out,
          ir0, ic0, ir1, ic1, ba0, bb0, ba1, bb1, pa0, pb0, pa1, pb1, pd,
          sem0, sem1, semi0, semi1):
        wid = lax.axis_index("s") * 2 + lax.axis_index("c")
        lbase0 = wid * epw
        slots = ((ir0, ic0, ba0, bb0, pa0, pb0, sem0, semi0),
                 (ir1, ic1, ba1, bb1, pa1, pb1, sem1, semi1))

        def load_idx(chunk, slot):
            ir, ic, ba, bb, pa, pb, sem, semi = slots[slot]
            cc = jnp.minimum(chunk, nchunk - 1)
            base = ebase + lbase0 + cc * _K
            pltpu.async_copy(row_hbm.at[pl.ds(base, _K)], ir, semi)
            pltpu.async_copy(col_hbm.at[pl.ds(base, _K)], ic, semi)

        def wait_idx(slot):
            ir, ic, ba, bb, pa, pb, sem, semi = slots[slot]
            pltpu.make_async_copy(row_hbm.at[pl.ds(0, _K)], ir, semi).wait()
            pltpu.make_async_copy(col_hbm.at[pl.ds(0, _K)], ic, semi).wait()

        def issue(slot):
            ir, ic, ba, bb, pa, pb, sem, semi = slots[slot]
            pltpu.async_copy(a_hbm.at[ir], ba, sem)
            pltpu.async_copy(b_hbm.at[ic], bb, sem)
            pltpu.async_copy(p_hbm.at[ir], pa, sem)
            pltpu.async_copy(p_hbm.at[ic], pb, sem)

        def drain(slot):
            ir, ic, ba, bb, pa, pb, sem, semi = slots[slot]
            pltpu.make_async_copy(a_hbm.at[ir], ba, sem).wait()
            pltpu.make_async_copy(b_hbm.at[ic], bb, sem).wait()
            pltpu.make_async_copy(p_hbm.at[ir], pa, sem).wait()
            pltpu.make_async_copy(p_hbm.at[ic], pb, sem).wait()

        def process(chunk, slot):
            ir, ic, ba, bb, pa, pb, sem, semi = slots[slot]
            lbase = lbase0 + chunk * _K

            def addrow(r, cc):
                for l in range(_D // 16):
                    sl = pl.ds(l * 16, 16)
                    ba[r, sl] = ba[r, sl] + bb[r, sl]
                s16 = pl.ds(0, 16)
                pd[r, s16] = pa[r, s16] - pb[r, s16]
                return cc

            lax.fori_loop(0, _K, addrow, 0)
            pltpu.sync_copy(ba, g_out.at[pl.ds(lbase, _K)])
            pltpu.sync_copy(pd, d_out.at[pl.ds(lbase, _K)])

        _pipeline3(nchunk, load_idx, wait_idx, issue, drain, process)

    return k(a, b, row, col, posp)


def _sc_gather(a, b, row, col, ebase, epw, nchunk):
    ne = _NW * epw

    @functools.partial(
        pl.kernel, mesh=_sc_mesh(),
        out_type=jax.ShapeDtypeStruct((ne, _D), _F32),
        scratch_types=[pltpu.VMEM((_K,), jnp.int32), pltpu.VMEM((_K,), jnp.int32),
                       pltpu.VMEM((_K,), jnp.int32), pltpu.VMEM((_K,), jnp.int32),
                       pltpu.VMEM((_K, _D), _F32), pltpu.VMEM((_K, _D), _F32),
                       pltpu.VMEM((_K, _D), _F32), pltpu.VMEM((_K, _D), _F32),
                       pltpu.SemaphoreType.DMA, pltpu.SemaphoreType.DMA,
                       pltpu.SemaphoreType.DMA, pltpu.SemaphoreType.DMA],
    )
    def k(a_hbm, b_hbm, row_hbm, col_hbm, g_out,
          ir0, ic0, ir1, ic1, ba0, bb0, ba1, bb1, sem0, sem1, semi0, semi1):
        wid = lax.axis_index("s") * 2 + lax.axis_index("c")
        lbase0 = wid * epw
        slots = ((ir0, ic0, ba0, bb0, sem0, semi0),
                 (ir1, ic1, ba1, bb1, sem1, semi1))

        def load_idx(chunk, slot):
            ir, ic, ba, bb, sem, semi = slots[slot]
            cc = jnp.minimum(chunk, nchunk - 1)
            base = ebase + lbase0 + cc * _K
            pltpu.async_copy(row_hbm.at[pl.ds(base, _K)], ir, semi)
            pltpu.async_copy(col_hbm.at[pl.ds(base, _K)], ic, semi)

        def wait_idx(slot):
            ir, ic, ba, bb, sem, semi = slots[slot]
            pltpu.make_async_copy(row_hbm.at[pl.ds(0, _K)], ir, semi).wait()
            pltpu.make_async_copy(col_hbm.at[pl.ds(0, _K)], ic, semi).wait()

        def issue(slot):
            ir, ic, ba, bb, sem, semi = slots[slot]
            pltpu.async_copy(a_hbm.at[ir], ba, sem)
            pltpu.async_copy(b_hbm.at[ic], bb, sem)

        def drain(slot):
            ir, ic, ba, bb, sem, semi = slots[slot]
            pltpu.make_async_copy(a_hbm.at[ir], ba, sem).wait()
            pltpu.make_async_copy(b_hbm.at[ic], bb, sem).wait()

        def process(chunk, slot):
            ir, ic, ba, bb, sem, semi = slots[slot]
            lbase = lbase0 + chunk * _K

            def addrow(r, cc):
                for l in range(_D // 16):
                    sl = pl.ds(l * 16, 16)
                    ba[r, sl] = ba[r, sl] + bb[r, sl]
                return cc

            lax.fori_loop(0, _K, addrow, 0)
            pltpu.sync_copy(ba, g_out.at[pl.ds(lbase, _K)])

        _pipeline3(nchunk, load_idx, wait_idx, issue, drain, process)

    return k(a, b, row, col)


def _sc_scatter(m, row, zeros, ebase, epw, nchunk):
    @functools.partial(
        pl.kernel, mesh=_sc_mesh(),
        out_type=jax.ShapeDtypeStruct((2, _N, _D), _F32),
        scratch_types=[pltpu.VMEM((_K,), jnp.int32), pltpu.VMEM((_K,), jnp.int32),
                       pltpu.VMEM((_K, _D), _F32), pltpu.VMEM((_K, _D), _F32),
                       pltpu.VMEM_SHARED((_N, _D), _F32),
                       pltpu.SemaphoreType.DMA, pltpu.SemaphoreType.DMA],
    )
    def k(m_hbm, row_hbm, z_hbm, out_hbm, idx0, idx1, buf0, buf1, acc,
          sem0, sem1):
        c = lax.axis_index("c")
        s = lax.axis_index("s")
        wid = s * 2 + c
        a0 = s * 624
        pltpu.sync_copy(z_hbm.at[pl.ds(a0, 640)], acc.at[pl.ds(a0, 640)])
        plsc.subcore_barrier()
        lbase0 = wid * epw
        slots = ((idx0, buf0, sem0), (idx1, buf1, sem1))

        def load(chunk, slot):
            idx, buf, sem = slots[slot]
            lbase = lbase0 + chunk * _K
            pltpu.async_copy(row_hbm.at[pl.ds(ebase + lbase, _K)], idx, sem)
            pltpu.async_copy(m_hbm.at[pl.ds(lbase, _K)], buf, sem)

        def drain(slot):
            idx, buf, sem = slots[slot]
            pltpu.make_async_copy(row_hbm.at[pl.ds(0, _K)], idx, sem).wait()
            pltpu.make_async_copy(m_hbm.at[pl.ds(0, _K)], buf, sem).wait()

        def process(chunk, slot):
            idx, buf, sem = slots[slot]
            pltpu.sync_copy(buf, acc.at[idx], add=True)

        _pipeline(nchunk, load, drain, process)
        plsc.subcore_barrier()
        pltpu.sync_copy(acc.at[pl.ds(a0, 640)],
                        out_hbm.at[c, pl.ds(a0, 640)])

    return k(m, row, zeros)


def kernel(x, pos, edge_index, mask, edge_attr, params):
    row = edge_index[0]
    col = edge_index[1]
    posp = jnp.pad(pos, ((0, 0), (0, _D - 3)))
    posp16 = posp[:, :16]
    zeros_d = jnp.zeros((_N, _D), _F32)

    def split(w):
        return w[:_D], w[_D:2 * _D], w[2 * _D:2 * _D + 1], w[2 * _D + 1:]

    def r1(v):
        return v[None, :]

    g0p = params['gcl0']
    g1p = params['gcl1']
    egp = params['egcl']
    wa0, wb0, wcr0, wce0 = split(g0p['ew1'])
    wa1, wb1, wcr1, wce1 = split(g1p['ew1'])
    wa2, wb2, wcr2, wce2 = split(egp['cw1'])
    w3p = jnp.pad(egp['cw3'], ((0, 0), (0, 7)))

    halves = ((0, _EPW0, _CH0, 0, _T0), (_EH0, _EPW1, _CH1, _T0, _T1))

    def edge_layer(a, b, wce, wcr, b1, w2, b2, diffs):
        qs = []
        for (ebase, epw, nchunk, toff, ntile), df in zip(halves, diffs):
            g = _sc_gather(a, b, row, col, ebase, epw, nchunk)
            m = _tc_edge(g, edge_attr, df, wce, wcr, b1, w2, b2, toff, ntile)
            q = _sc_scatter(m, row, zeros_d, ebase, epw, nchunk)
            qs += [q[0], q[1]]
        return qs

    a0, b0 = _tc_ab(x, wa0, wb0)
    qs0 = []
    diffs = []
    for ebase, epw, nchunk, toff, ntile in halves:
        g, df = _sc_gather_pos(a0, b0, row, col, posp, ebase, epw, nchunk)
        diffs.append(df)
        m = _tc_edge(g, edge_attr, df, wce0, wcr0, r1(g0p['eb1']),
                     g0p['ew2'], r1(g0p['eb2']), toff, ntile)
        q = _sc_scatter(m, row, zeros_d, ebase, epw, nchunk)
        qs0 += [q[0], q[1]]
    h1, a1, b1 = _tc_node(x, *qs0,
                          g0p['nw1'][:_D], g0p['nw1'][_D:], r1(g0p['nb1']),
                          g0p['nw2'], r1(g0p['nb2']), wa1, wb1)

    qs1 = edge_layer(a1, b1, wce1, wcr1, r1(g1p['eb1']),
                     g1p['ew2'], r1(g1p['eb2']), diffs)
    h2, a2, b2 = _tc_node(h1, *qs1,
                          g1p['nw1'][:_D], g1p['nw1'][_D:], r1(g1p['nb1']),
                          g1p['nw2'], r1(g1p['nb2']), wa2, wb2)

    qsp = []
    for (ebase, epw, nchunk, toff, ntile), df in zip(halves, diffs):
        g = _sc_gather(a2, b2, row, col, ebase, epw, nchunk)
        t = _tc_coord(g, edge_attr, df, wce2, wcr2, r1(egp['cb1']),
                      egp['cw2'], r1(egp['cb2']), w3p, toff, ntile)
        q = _sc_scatter(t, row, zeros_d, ebase, epw, nchunk)
        qsp += [q[0, :, :16], q[1, :, :16]]
    pos16 = _tc_pos(posp16, *qsp, mask)
    return h2, pos16[:, :3]
